# Initial kernel scaffold; baseline (speedup 1.0000x reference)
import jax
import jax.numpy as jnp
from jax.experimental import pallas as pl


def kernel(x, edge_index, edge_weights, W_w, b_w, att):
    raise NotImplementedError("write your pallas kernel here")


# speedup vs baseline: 74.7800x; 74.7800x over previous
import functools

import jax
import jax.numpy as jnp
from jax import lax
from jax.experimental import pallas as pl
from jax.experimental.pallas import tpu as pltpu
from jax.experimental.pallas import tpu_sc as plsc

NC = 2
NS = 16
NW = NC * NS
LANES = 16
CHUNK = 128


def _sc_partials(x, srcm, dstm, n_pad, rpt):
    d = x.shape[1]
    nch = srcm.shape[1]
    rowb = CHUNK

    mesh = plsc.VectorSubcoreMesh(core_axis_name="c", subcore_axis_name="s")

    @functools.partial(
        pl.kernel,
        out_type=jax.ShapeDtypeStruct((NC, n_pad, d), jnp.float32),
        mesh=mesh,
        scratch_types=[
            pltpu.VMEM_SHARED((n_pad, d), jnp.float32),
            pltpu.VMEM((nch, CHUNK), jnp.int32),
            pltpu.VMEM((nch, CHUNK), jnp.int32),
            pltpu.VMEM((2, rowb, d), jnp.float32),
            pltpu.SemaphoreType.DMA,
            pltpu.SemaphoreType.DMA,
        ],
    )
    def k(x_hbm, srcm_hbm, dstm_hbm, out_hbm, acc, src_v, dst_v, rows_v, sem0, sem1):
        cid = lax.axis_index("c")
        sid = lax.axis_index("s")
        wid = sid * NC + cid

        pltpu.sync_copy(srcm_hbm.at[wid], src_v)
        pltpu.sync_copy(dstm_hbm.at[wid], dst_v)

        zero16 = jnp.zeros((LANES,), jnp.float32)

        @pl.loop(0, rowb)
        def _zero_rows(r):
            for c in range(d // LANES):
                rows_v[0, r, pl.ds(c * LANES, LANES)] = zero16

        base = sid * rpt
        for q in range(rpt // rowb):
            pltpu.sync_copy(rows_v.at[0], acc.at[pl.ds(base + q * rowb, rowb)])
        plsc.subcore_barrier()

        sems = (sem0, sem1)
        pltpu.async_copy(x_hbm.at[src_v.at[0]], rows_v.at[0], sem0)
        pltpu.async_copy(x_hbm.at[src_v.at[1]], rows_v.at[1], sem1)

        @pl.loop(0, nch // 2)
        def _pairs(g):
            for b in range(2):
                j = g * 2 + b
                buf = rows_v.at[b]
                pltpu.make_async_copy(x_hbm.at[src_v.at[j]], buf, sems[b]).wait()
                pltpu.sync_copy(buf, acc.at[dst_v.at[j]], add=True)

                @pl.when(j + 2 < nch)
                def _fire():
                    pltpu.async_copy(x_hbm.at[src_v.at[j + 2]], buf, sems[b])

        plsc.subcore_barrier()

        pltpu.sync_copy(acc.at[pl.ds(base, rpt)], out_hbm.at[cid].at[pl.ds(base, rpt)])

    return k


def _combine_body(x_ref, p_ref, o_ref, *, scale):
    o_ref[...] = jnp.maximum((x_ref[...] + p_ref[0] + p_ref[1]) * scale, 0.0)


def kernel(x, edge_index, edge_weights, W_w, b_w, att):
    n, d = x.shape
    e = edge_index.shape[1]
    out_dim = att.shape[1]

    src = edge_index[0].astype(jnp.int32)
    dst = edge_index[1].astype(jnp.int32)

    ept = -(-e // (NW * CHUNK)) * CHUNK
    pad = NW * ept - e
    src_p = jnp.concatenate([src, jnp.zeros((pad,), jnp.int32)])
    dst_p = jnp.concatenate([dst, jnp.full((pad,), n, jnp.int32)])
    srcm = src_p.reshape(NW, ept // CHUNK, CHUNK)
    dstm = dst_p.reshape(NW, ept // CHUNK, CHUNK)

    rpt = -(-(-(-n // NS)) // CHUNK) * CHUNK
    n_pad = NS * rpt
    assert n_pad > n

    partials = _sc_partials(x, srcm, dstm, n_pad, rpt)(x, srcm, dstm)

    blk = 1000
    out = pl.pallas_call(
        functools.partial(_combine_body, scale=1.0 / out_dim),
        out_shape=jax.ShapeDtypeStruct((n, d), jnp.float32),
        grid=(n // blk,),
        in_specs=[
            pl.BlockSpec((blk, d), lambda i: (i, 0)),
            pl.BlockSpec((NC, blk, d), lambda i: (0, i, 0)),
        ],
        out_specs=pl.BlockSpec((blk, d), lambda i: (i, 0)),
    )(x, partials)
    return out


# speedup vs baseline: 74.8868x; 1.0014x over previous
import functools

import jax
import jax.numpy as jnp
from jax import lax
from jax.experimental import pallas as pl
from jax.experimental.pallas import tpu as pltpu
from jax.experimental.pallas import tpu_sc as plsc

NC = 2
NS = 16
NW = NC * NS
LANES = 16
CHUNK = 128


def _sc_partials(x, srcm, dstm, n_pad, rpt):
    d = x.shape[1]
    nch = srcm.shape[1]
    rowb = CHUNK

    mesh = plsc.VectorSubcoreMesh(core_axis_name="c", subcore_axis_name="s")

    @functools.partial(
        pl.kernel,
        out_type=jax.ShapeDtypeStruct((NC, n_pad, d), jnp.float32),
        mesh=mesh,
        scratch_types=[
            pltpu.VMEM_SHARED((n_pad, d), jnp.float32),
            pltpu.VMEM((nch, CHUNK), jnp.int32),
            pltpu.VMEM((nch, CHUNK), jnp.int32),
            pltpu.VMEM((2, rowb, d), jnp.float32),
            pltpu.SemaphoreType.DMA,
            pltpu.SemaphoreType.DMA,
        ],
    )
    def k(x_hbm, srcm_hbm, dstm_hbm, out_hbm, acc, src_v, dst_v, rows_v, sem0, sem1):
        cid = lax.axis_index("c")
        sid = lax.axis_index("s")
        wid = sid * NC + cid

        pltpu.sync_copy(srcm_hbm.at[wid], src_v)
        pltpu.sync_copy(dstm_hbm.at[wid], dst_v)

        zero16 = jnp.zeros((LANES,), jnp.float32)

        @pl.loop(0, rowb)
        def _zero_rows(r):
            for c in range(d // LANES):
                rows_v[0, r, pl.ds(c * LANES, LANES)] = zero16

        base = sid * rpt
        for q in range(rpt // rowb):
            pltpu.sync_copy(rows_v.at[0], acc.at[pl.ds(base + q * rowb, rowb)])
        plsc.subcore_barrier()

        sems = (sem0, sem1)
        pltpu.async_copy(x_hbm.at[src_v.at[0]], rows_v.at[0], sem0)
        pltpu.async_copy(x_hbm.at[src_v.at[1]], rows_v.at[1], sem1)

        @pl.loop(0, nch // 2)
        def _pairs(g):
            for b in range(2):
                j = g * 2 + b
                buf = rows_v.at[b]
                pltpu.make_async_copy(x_hbm.at[src_v.at[j]], buf, sems[b]).wait()
                pltpu.sync_copy(buf, acc.at[dst_v.at[j]], add=True)

                @pl.when(j + 2 < nch)
                def _fire():
                    pltpu.async_copy(x_hbm.at[src_v.at[j + 2]], buf, sems[b])

        plsc.subcore_barrier()

        pltpu.sync_copy(acc.at[pl.ds(base, rpt)], out_hbm.at[cid].at[pl.ds(base, rpt)])

    return k


def _combine_body(x_ref, p_ref, o_ref, *, scale):
    o_ref[...] = jnp.maximum((x_ref[...] + p_ref[0] + p_ref[1]) * scale, 0.0)


def kernel(x, edge_index, edge_weights, W_w, b_w, att):
    n, d = x.shape
    e = edge_index.shape[1]
    out_dim = att.shape[1]

    src = edge_index[0].astype(jnp.int32)
    dst = edge_index[1].astype(jnp.int32)

    ept = -(-e // (NW * CHUNK)) * CHUNK
    pad = NW * ept - e

    rpt = -(-(-(-n // NS)) // CHUNK) * CHUNK
    n_pad = NS * rpt
    assert n_pad > n

    src_p = jnp.concatenate([src, jnp.zeros((pad,), jnp.int32)])
    dst_p = jnp.concatenate([dst, n + (jnp.arange(pad, dtype=jnp.int32) % (n_pad - n))])
    srcm = src_p.reshape(NW, ept // CHUNK, CHUNK)
    dstm = dst_p.reshape(NW, ept // CHUNK, CHUNK)

    partials = _sc_partials(x, srcm, dstm, n_pad, rpt)(x, srcm, dstm)

    blk = 1000
    out = pl.pallas_call(
        functools.partial(_combine_body, scale=1.0 / out_dim),
        out_shape=jax.ShapeDtypeStruct((n, d), jnp.float32),
        grid=(n // blk,),
        in_specs=[
            pl.BlockSpec((blk, d), lambda i: (i, 0)),
            pl.BlockSpec((NC, blk, d), lambda i: (0, i, 0)),
        ],
        out_specs=pl.BlockSpec((blk, d), lambda i: (i, 0)),
    )(x, partials)
    return out
